# baseline (speedup 1.0000x reference)
import functools

import jax
import jax.numpy as jnp
from jax import lax
from jax.experimental import pallas as pl
from jax.experimental.pallas import tpu as pltpu
from jax.experimental.pallas import tpu_sc as plsc

NC = 2
NS = 16
NW = NC * NS

CHUNK = 512
GATHER = 128
TPC = CHUNK // 128


def _count_zeros(idx_v):
    ones = jnp.ones((16,), jnp.int32)
    zer = jnp.zeros((16,), jnp.int32)

    def cnt_body(i, acc):
        v = idx_v[pl.ds(i * 16, 16)]
        return acc + jnp.sum(jnp.where(v == 0, ones, zer))

    return lax.fori_loop(0, CHUNK // 16, cnt_body, jnp.int32(0))


def _fix_zero_rows(E, idx_v, rows_v):
    zeros = jnp.zeros((16,), jnp.float32)

    def fix_body(i, carry):
        v = idx_v[pl.ds(i * 16, 16)]
        m = v == 0
        rowids = lax.iota(jnp.int32, 16) + i * 16
        for col in range(E):
            colids = jnp.full((16,), col, jnp.int32)
            plsc.store_scatter(rows_v, [rowids, colids], zeros, mask=m)
        return carry

    lax.fori_loop(0, CHUNK // 16, fix_body, jnp.int32(0))


def _transpose_chunk(rows_v, phys_v):
    lanes = lax.iota(jnp.int32, 16)
    lane_k = [lanes + 16 * k for k in range(8)]

    for tr in range(4):
        def body(j, carry, tr=tr):
            rows_k = [j * 128 + lk for lk in lane_k]
            for fe in range(8):
                col = jnp.full((16,), 8 * tr + fe, jnp.int32)
                for k in range(8):
                    v = plsc.load_gather(rows_v, [rows_k[k], col])
                    phys_v[tr, j, fe, pl.ds(16 * k, 16)] = v
            return carry

        lax.fori_loop(0, TPC, body, jnp.int32(0))


def _gather_body(E, cpl, nch, idx_hbm, table_hbm, out_hbm,
                 idx0, idx1, rows0, rows1, phys0, phys1,
                 sg0, sg1, sw0, sw1):
    wid = lax.axis_index("s") * NC + lax.axis_index("c")
    cbase = wid * nch

    def load_and_fire(c, ib, rb, sg):
        pltpu.sync_copy(idx_hbm.at[pl.ds((cbase + c) * CHUNK, CHUNK)], ib)
        for j in range(CHUNK // GATHER):
            pltpu.async_copy(
                table_hbm.at[ib.at[pl.ds(j * GATHER, GATHER)]],
                rb.at[pl.ds(j * GATHER, GATHER)], sg)

    def wait_gathers(rb, sg):
        pltpu.make_async_copy(table_hbm.at[pl.ds(0, CHUNK)], rb, sg).wait()

    def wait_write(pv, sw):
        pltpu.make_async_copy(
            pv, out_hbm.at[0, :, pl.ds(0, TPC), :, :], sw).wait()

    def process(c, nz, ib, rb, pv, sg, sw):
        wait_gathers(rb, sg)

        @pl.when(nz > 0)
        def _():
            _fix_zero_rows(E, ib, rb)

        _transpose_chunk(rb, pv)
        cg = cbase + c
        l = cg // cpl
        tc0 = (cg % cpl) * TPC
        pltpu.async_copy(pv, out_hbm.at[l, :, pl.ds(tc0, TPC), :, :], sw)

    def prefetch(c, first, last, ib, rb, sg, pv, sw):
        @pl.when(jnp.logical_not(jnp.logical_or(first, last)))
        def _():
            wait_write(pv, sw)

        @pl.when(jnp.logical_not(last))
        def _():
            load_and_fire(c, ib, rb, sg)
        return _count_zeros(ib)

    nz0 = prefetch(0, jnp.bool_(True), jnp.bool_(False), idx0, rows0,
                   sg0, phys0, sw0)

    def loop_body(i, carry):
        nz0, nz1 = carry
        c0 = 2 * i
        nz1 = prefetch(c0 + 1, i == 0, jnp.bool_(False), idx1, rows1,
                       sg1, phys1, sw1)
        process(c0, nz0, idx0, rows0, phys0, sg0, sw0)
        nz0 = prefetch(c0 + 2, jnp.bool_(False), i == nch // 2 - 1,
                       idx0, rows0, sg0, phys0, sw0)
        process(c0 + 1, nz1, idx1, rows1, phys1, sg1, sw1)
        return nz0, nz1

    lax.fori_loop(0, nch // 2, loop_body, (nz0, nz0))

    wait_write(phys0, sw0)
    wait_write(phys1, sw1)


def kernel(input, table, max_steps):
    B, L = input.shape
    V, E = table.shape
    N = B * L
    cpl = B // CHUNK
    nch = N // CHUNK // NW

    idx_flat = input.T.reshape(N).astype(jnp.int32)

    mesh = plsc.VectorSubcoreMesh(core_axis_name="c", subcore_axis_name="s")
    body = functools.partial(_gather_body, E, cpl, nch)
    out5 = pl.kernel(
        body,
        out_type=jax.ShapeDtypeStruct((L, E // 8, B // 128, 8, 128),
                                      jnp.float32),
        mesh=mesh,
        compiler_params=pltpu.CompilerParams(use_tc_tiling_on_sc=False,
                                             needs_layout_passes=False),
        scratch_types=[
            pltpu.VMEM((CHUNK,), jnp.int32),
            pltpu.VMEM((CHUNK,), jnp.int32),
            pltpu.VMEM((CHUNK, E), jnp.float32),
            pltpu.VMEM((CHUNK, E), jnp.float32),
            pltpu.VMEM((E // 8, TPC, 8, 128), jnp.float32),
            pltpu.VMEM((E // 8, TPC, 8, 128), jnp.float32),
            pltpu.SemaphoreType.DMA,
            pltpu.SemaphoreType.DMA,
            pltpu.SemaphoreType.DMA,
            pltpu.SemaphoreType.DMA,
        ],
    )(idx_flat, table)
    return out5.transpose(0, 2, 4, 1, 3).reshape(L, B, E)


# speedup vs baseline: 1.7207x; 1.7207x over previous
import functools

import jax
import jax.numpy as jnp
from jax import lax
from jax.experimental import pallas as pl
from jax.experimental.pallas import tpu as pltpu
from jax.experimental.pallas import tpu_sc as plsc

NC = 2
NS = 16
NW = NC * NS

CHUNK = 512
GATHER = 128
PADW = 128


def _count_zeros(idx_v):
    ones = jnp.ones((16,), jnp.int32)
    zer = jnp.zeros((16,), jnp.int32)

    def cnt_body(i, acc):
        v = idx_v[pl.ds(i * 16, 16)]
        return acc + jnp.sum(jnp.where(v == 0, ones, zer))

    return lax.fori_loop(0, CHUNK // 16, cnt_body, jnp.int32(0))


def _fix_zero_rows(E, idx_v, rows_v):
    zeros = jnp.zeros((16,), jnp.float32)

    def fix_body(i, carry):
        v = idx_v[pl.ds(i * 16, 16)]
        m = v == 0
        rowids = lax.iota(jnp.int32, 16) + i * 16
        for col in range(E):
            colids = jnp.full((16,), col, jnp.int32)
            plsc.store_scatter(rows_v, [rowids, colids], zeros, mask=m)
        return carry

    lax.fori_loop(0, CHUNK // 16, fix_body, jnp.int32(0))


def _gather_body(E, cpl, nch, idx_hbm, table_hbm, out_hbm,
                 idx0, idx1, rows0, rows1, sg0, sg1, sw0, sw1):
    wid = lax.axis_index("s") * NC + lax.axis_index("c")
    cbase = wid * nch

    def load_and_fire(c, ib, rb, sg):
        pltpu.sync_copy(idx_hbm.at[pl.ds((cbase + c) * CHUNK, CHUNK)], ib)
        for j in range(CHUNK // GATHER):
            pltpu.async_copy(
                table_hbm.at[ib.at[pl.ds(j * GATHER, GATHER)]],
                rb.at[pl.ds(j * GATHER, GATHER)], sg)

    def wait_gathers(rb, sg):
        pltpu.make_async_copy(table_hbm.at[pl.ds(0, CHUNK)], rb, sg).wait()

    def wait_write(rb, sw):
        pltpu.make_async_copy(table_hbm.at[pl.ds(0, CHUNK)], rb, sw).wait()

    def process(c, nz, ib, rb, sg, sw):
        wait_gathers(rb, sg)

        @pl.when(nz > 0)
        def _():
            _fix_zero_rows(E, ib, rb)

        cg = cbase + c
        l = cg // cpl
        b0 = (cg % cpl) * CHUNK
        pltpu.async_copy(rb, out_hbm.at[l, pl.ds(b0, CHUNK), pl.ds(0, E)],
                         sw)

    def prefetch(c, first, last, ib, rb, sg, sw):
        @pl.when(jnp.logical_not(jnp.logical_or(first, last)))
        def _():
            wait_write(rb, sw)

        @pl.when(jnp.logical_not(last))
        def _():
            load_and_fire(c, ib, rb, sg)
        return _count_zeros(ib)

    nz0 = prefetch(0, jnp.bool_(True), jnp.bool_(False), idx0, rows0,
                   sg0, sw0)

    def loop_body(i, carry):
        nz0, nz1 = carry
        c0 = 2 * i
        nz1 = prefetch(c0 + 1, i == 0, jnp.bool_(False), idx1, rows1,
                       sg1, sw1)
        process(c0, nz0, idx0, rows0, sg0, sw0)
        nz0 = prefetch(c0 + 2, jnp.bool_(False), i == nch // 2 - 1,
                       idx0, rows0, sg0, sw0)
        process(c0 + 1, nz1, idx1, rows1, sg1, sw1)
        return nz0, nz1

    lax.fori_loop(0, nch // 2, loop_body, (nz0, nz0))

    wait_write(rows0, sw0)
    wait_write(rows1, sw1)


def kernel(input, table, max_steps):
    B, L = input.shape
    V, E = table.shape
    N = B * L
    cpl = B // CHUNK
    nch = N // CHUNK // NW

    idx_flat = input.T.reshape(N).astype(jnp.int32)

    mesh = plsc.VectorSubcoreMesh(core_axis_name="c", subcore_axis_name="s")
    body = functools.partial(_gather_body, E, cpl, nch)
    out = pl.kernel(
        body,
        out_type=jax.ShapeDtypeStruct((L, B, PADW), jnp.float32),
        mesh=mesh,
        compiler_params=pltpu.CompilerParams(use_tc_tiling_on_sc=False,
                                             needs_layout_passes=False),
        scratch_types=[
            pltpu.VMEM((CHUNK,), jnp.int32),
            pltpu.VMEM((CHUNK,), jnp.int32),
            pltpu.VMEM((CHUNK, E), jnp.float32),
            pltpu.VMEM((CHUNK, E), jnp.float32),
            pltpu.SemaphoreType.DMA,
            pltpu.SemaphoreType.DMA,
            pltpu.SemaphoreType.DMA,
            pltpu.SemaphoreType.DMA,
        ],
    )(idx_flat, table)
    return out[:, :, :E]


# speedup vs baseline: 1.8118x; 1.0529x over previous
import functools

import jax
import jax.numpy as jnp
from jax import lax
from jax.experimental import pallas as pl
from jax.experimental.pallas import tpu as pltpu
from jax.experimental.pallas import tpu_sc as plsc

NC = 2
NS = 16
NW = NC * NS

CHUNK = 512
GATHER = 128
PADW = 128

GCOLS = 512
NG = 61


def _transpose_group(in_v, out_v, ncol):
    lanes = lax.iota(jnp.int32, 16)
    rowk = [4 * k + (lanes >> 2) for k in range(ncol // 16)]
    colbase = (lanes & 3) * 32

    @plsc.parallel_loop(0, 32)
    def _(fe):
        colv = colbase + fe
        for k in range(ncol // 16):
            v = in_v[fe, pl.ds(16 * k, 16)]
            plsc.store_scatter(out_v, [rowk[k], colv], v)


def _retile_body(V, tT_hbm, trm_hbm, in0, in1, out0, out1, inp, outp,
                 si0, si1, so0, so1, sp):
    wid = lax.axis_index("s") * NC + lax.axis_index("c")
    vfull = (V // GCOLS) * GCOLS

    def gcol(t):
        return (wid + 32 * t) * GCOLS

    def load(t, iv, si):
        pltpu.async_copy(tT_hbm.at[:, pl.ds(gcol(t), GCOLS)], iv, si)

    def wait_load(iv, si):
        pltpu.make_async_copy(tT_hbm.at[:, pl.ds(0, GCOLS)], iv, si).wait()

    def wait_write(ov, so):
        pltpu.make_async_copy(trm_hbm.at[pl.ds(0, GCOLS // 4), :], ov,
                              so).wait()

    def process(t, iv, ov, si, so):
        wait_load(iv, si)
        _transpose_group(iv, ov, GCOLS)
        pltpu.async_copy(
            ov, trm_hbm.at[pl.ds((gcol(t) // GCOLS) * (GCOLS // 4),
                                 GCOLS // 4), :], so)

    load(0, in0, si0)

    def loop_body(i, carry):
        t0 = 2 * i
        load(t0 + 1, in1, si1)

        @pl.when(i > 0)
        def _():
            wait_write(out0, so0)
        process(t0, in0, out0, si0, so0)

        @pl.when(t0 + 2 < NG)
        def _():
            load(t0 + 2, in0, si0)

        @pl.when(i > 0)
        def _():
            wait_write(out1, so1)
        process(t0 + 1, in1, out1, si1, so1)
        return carry

    lax.fori_loop(0, NG // 2, loop_body, jnp.int32(0))
    wait_write(out0, so0)
    process(NG - 1, in0, out0, si0, so0)
    wait_write(out0, so0)
    wait_write(out1, so1)

    @pl.when(wid == 0)
    def _():
        pltpu.sync_copy(tT_hbm.at[:, pl.ds(vfull - GCOLS, GCOLS)], in0)
        _transpose_group(in0, out0, GCOLS)
        pltpu.sync_copy(
            out0, trm_hbm.at[pl.ds((vfull - GCOLS) // 4, GCOLS // 4), :])

        tail = V - vfull
        pltpu.sync_copy(tT_hbm.at[:, pl.ds(vfull, tail)], inp)
        _transpose_group(inp, outp, tail)
        pltpu.sync_copy(outp, trm_hbm.at[pl.ds(vfull // 4, tail // 4), :])


def _count_zeros(idx_v):
    ones = jnp.ones((16,), jnp.int32)
    zer = jnp.zeros((16,), jnp.int32)

    def cnt_body(i, acc):
        v = idx_v[pl.ds(i * 16, 16)]
        return acc + jnp.sum(jnp.where(v == 0, ones, zer))

    return lax.fori_loop(0, CHUNK // 16, cnt_body, jnp.int32(0))


def _fix_zero_rows(E, idx_v, rows_v):
    zeros = jnp.zeros((16,), jnp.float32)

    def fix_body(i, carry):
        v = idx_v[pl.ds(i * 16, 16)]
        m = v == 0
        rowids = lax.iota(jnp.int32, 16) + i * 16
        for col in range(E):
            colids = jnp.full((16,), col, jnp.int32)
            plsc.store_scatter(rows_v, [rowids, colids], zeros, mask=m)
        return carry

    lax.fori_loop(0, CHUNK // 16, fix_body, jnp.int32(0))


def _gather_body(E, cpl, nch, idx_hbm, table_hbm, out_hbm,
                 idx0, idx1, rows0, rows1, sg0, sg1, sw0, sw1):
    wid = lax.axis_index("s") * NC + lax.axis_index("c")
    cbase = wid * nch

    def load_and_fire(c, ib, rb, sg):
        pltpu.sync_copy(idx_hbm.at[pl.ds((cbase + c) * CHUNK, CHUNK)], ib)
        for j in range(CHUNK // GATHER):
            pltpu.async_copy(
                table_hbm.at[ib.at[pl.ds(j * GATHER, GATHER)]],
                rb.at[pl.ds(j * GATHER, GATHER)], sg)

    def wait_gathers(rb, sg):
        pltpu.make_async_copy(table_hbm.at[pl.ds(0, CHUNK)], rb, sg).wait()

    def wait_write(rb, sw):
        pltpu.make_async_copy(table_hbm.at[pl.ds(0, CHUNK)], rb, sw).wait()

    def process(c, nz, ib, rb, sg, sw):
        wait_gathers(rb, sg)

        @pl.when(nz > 0)
        def _():
            _fix_zero_rows(E, ib, rb)

        cg = cbase + c
        l = cg // cpl
        b0 = (cg % cpl) * CHUNK
        pltpu.async_copy(rb, out_hbm.at[l, pl.ds(b0, CHUNK), pl.ds(0, E)],
                         sw)

    def prefetch(c, first, last, ib, rb, sg, sw):
        @pl.when(jnp.logical_not(jnp.logical_or(first, last)))
        def _():
            wait_write(rb, sw)

        @pl.when(jnp.logical_not(last))
        def _():
            load_and_fire(c, ib, rb, sg)
        return _count_zeros(ib)

    nz0 = prefetch(0, jnp.bool_(True), jnp.bool_(False), idx0, rows0,
                   sg0, sw0)

    def loop_body(i, carry):
        nz0, nz1 = carry
        c0 = 2 * i
        nz1 = prefetch(c0 + 1, i == 0, jnp.bool_(False), idx1, rows1,
                       sg1, sw1)
        process(c0, nz0, idx0, rows0, sg0, sw0)
        nz0 = prefetch(c0 + 2, jnp.bool_(False), i == nch // 2 - 1,
                       idx0, rows0, sg0, sw0)
        process(c0 + 1, nz1, idx1, rows1, sg1, sw1)
        return nz0, nz1

    lax.fori_loop(0, nch // 2, loop_body, (nz0, nz0))

    wait_write(rows0, sw0)
    wait_write(rows1, sw1)


def kernel(input, table, max_steps):
    B, L = input.shape
    V, E = table.shape
    N = B * L
    cpl = B // CHUNK
    nch = N // CHUNK // NW

    idx_flat = input.T.reshape(N).astype(jnp.int32)

    mesh = plsc.VectorSubcoreMesh(core_axis_name="c", subcore_axis_name="s")

    trm = pl.kernel(
        functools.partial(_retile_body, V),
        out_type=jax.ShapeDtypeStruct((V // 4, 128), jnp.float32),
        mesh=mesh,
        compiler_params=pltpu.CompilerParams(use_tc_tiling_on_sc=True,
                                             needs_layout_passes=False),
        scratch_types=[
            pltpu.VMEM((32, GCOLS), jnp.float32),
            pltpu.VMEM((32, GCOLS), jnp.float32),
            pltpu.VMEM((GCOLS // 4, 128), jnp.float32),
            pltpu.VMEM((GCOLS // 4, 128), jnp.float32),
            pltpu.VMEM((32, 64), jnp.float32),
            pltpu.VMEM((16, 128), jnp.float32),
            pltpu.SemaphoreType.DMA,
            pltpu.SemaphoreType.DMA,
            pltpu.SemaphoreType.DMA,
            pltpu.SemaphoreType.DMA,
            pltpu.SemaphoreType.DMA,
        ],
    )(table.T)
    table_rm = trm.reshape(V, E)

    body = functools.partial(_gather_body, E, cpl, nch)
    out = pl.kernel(
        body,
        out_type=jax.ShapeDtypeStruct((L, B, PADW), jnp.float32),
        mesh=mesh,
        compiler_params=pltpu.CompilerParams(use_tc_tiling_on_sc=False,
                                             needs_layout_passes=False),
        scratch_types=[
            pltpu.VMEM((CHUNK,), jnp.int32),
            pltpu.VMEM((CHUNK,), jnp.int32),
            pltpu.VMEM((CHUNK, E), jnp.float32),
            pltpu.VMEM((CHUNK, E), jnp.float32),
            pltpu.SemaphoreType.DMA,
            pltpu.SemaphoreType.DMA,
            pltpu.SemaphoreType.DMA,
            pltpu.SemaphoreType.DMA,
        ],
    )(idx_flat, table_rm)
    return out[:, :, :E]
